# baseline (speedup 1.0000x reference)
import functools

import jax
import jax.numpy as jnp
from jax import lax
from jax.experimental import pallas as pl
from jax.experimental.pallas import tpu as pltpu
from jax.experimental.pallas import tpu_sc as plsc

NC = 2
NS = 16
NW = NC * NS
LANES = 16
KP = 16
CH = 1024
G = 128
NG = CH // G


def _project_table(emb_table, fc_wp):
    V, D = emb_table.shape
    grid = 1
    for cand in (8, 4, 2):
        if (V // 8) % cand == 0:
            grid = cand
            break
    rows = V // 8 // grid
    e8 = emb_table.reshape(grid, rows, 8 * D)
    wb = jnp.kron(jnp.eye(8, dtype=jnp.float32), fc_wp)

    def mm_body(x_ref, w_ref, o_ref):
        y = jnp.dot(x_ref[0], w_ref[...], preferred_element_type=jnp.float32)
        o_ref[...] = y.reshape(1, rows, 8 * KP)

    p3 = pl.pallas_call(
        mm_body,
        grid=(grid,),
        in_specs=[
            pl.BlockSpec((1, rows, 8 * D), lambda i: (i, 0, 0)),
            pl.BlockSpec((8 * D, 8 * KP), lambda i: (0, 0)),
        ],
        out_specs=pl.BlockSpec((1, rows, 8 * KP), lambda i: (i, 0, 0)),
        out_shape=jax.ShapeDtypeStruct((grid, rows, 8 * KP), jnp.float32),
    )(e8, wb)
    return p3.reshape(V, KP)


def _bag_body(bpw, offw, n_tok, text_hbm, offs_hbm, p_hbm, bias_hbm, out_hbm,
              off_v, idx0, idx1, rows0, rows1, out_v, bias_v, sem_a, sem_b):
    wid = lax.axis_index("s") * NC + lax.axis_index("c")
    base_bag = wid * bpw
    pltpu.sync_copy(offs_hbm.at[pl.ds(base_bag, offw)], off_v)
    pltpu.sync_copy(bias_hbm, bias_v)
    bias = bias_v[...]
    zero = jnp.zeros((LANES,), jnp.float32)

    s0 = off_v[pl.ds(0, LANES)][0]
    s1 = off_v[pl.ds(bpw, LANES)][0]
    a = jnp.bitwise_and(s0, jnp.int32(-8))
    nch = (s1 - a + (CH - 1)) // CH
    nch2 = ((nch + 1) // 2) * 2

    nvec = offw // LANES
    lane_iota = lax.iota(jnp.int32, LANES)

    def count_le(hi):
        m = jnp.int32(0)
        for k in range(nvec):
            valid = bpw + 1 - k * LANES
            if valid <= 0:
                break
            vk = off_v[pl.ds(k * LANES, LANES)]
            sel = (vk <= hi) & (lane_iota < valid)
            m = m + plsc.all_reduce_population_count(sel)[0]
        return m

    def chunk_start(j):
        c0 = jnp.minimum(a + j * CH, n_tok - CH)
        return pl.multiple_of(c0, 8)

    def prefetch(j, idx_b, rows_b, sem):
        c0 = chunk_start(j)
        pltpu.sync_copy(text_hbm.at[pl.ds(c0, CH)], idx_b)
        for g in range(NG):
            pltpu.async_copy(
                p_hbm.at[idx_b.at[pl.ds(g * G, G)]],
                rows_b.at[pl.ds(g * G, G)], sem)

    def drain(rows_b, sem):
        pltpu.make_async_copy(p_hbm.at[pl.ds(0, CH)], rows_b, sem).wait()

    def make_run_sum(rows_b):
        def run_sum(t0r, t1r, acc):
            n = jnp.maximum(t1r - t0r, 0)

            def tok8(i, st):
                a0, a1, a2, a3 = st
                b = t0r + i * 8
                a0 = a0 + rows_b[b, :]
                a1 = a1 + rows_b[b + 1, :]
                a2 = a2 + rows_b[b + 2, :]
                a3 = a3 + rows_b[b + 3, :]
                a0 = a0 + rows_b[b + 4, :]
                a1 = a1 + rows_b[b + 5, :]
                a2 = a2 + rows_b[b + 6, :]
                a3 = a3 + rows_b[b + 7, :]
                return a0, a1, a2, a3

            a0, a1, a2, a3 = lax.fori_loop(
                0, n // 8, tok8, (acc, zero, zero, zero))
            acc = (a0 + a1) + (a2 + a3)

            def tok1(i, a2_):
                return a2_ + rows_b[i, :]

            n8 = jnp.bitwise_and(n, jnp.int32(-8))
            return lax.fori_loop(t0r + n8, t1r, tok1, acc)

        return run_sum

    def process_chunk(j, rows_b, carry):
        t, lb, acc = carry
        run_sum = make_run_sum(rows_b)
        c0 = chunk_start(j)
        hi = jnp.minimum(c0 + CH, s1)
        nd = count_le(hi) - 1

        def bag_done(k, st):
            t, acc = st
            bv = off_v[pl.ds(k, LANES)]
            b_start, e_true = bv[0], bv[1]
            acc = run_sum(t - c0, e_true - c0, acc)
            cnt = (e_true - b_start).astype(jnp.float32)
            cnt_vec = jnp.full((LANES,), 1.0, jnp.float32) * cnt
            out_v[k, :] = acc / jnp.maximum(cnt_vec, 1.0) + bias
            return e_true, zero

        t, acc = lax.fori_loop(lb, nd, bag_done, (jnp.maximum(t, c0), acc))
        acc = run_sum(jnp.maximum(t, c0) - c0, hi - c0, acc)
        return hi, nd, acc

    prefetch(0, idx0, rows0, sem_a)

    def pair_body(jj, carry):
        j0 = 2 * jj
        prefetch(j0 + 1, idx1, rows1, sem_b)
        drain(rows0, sem_a)
        carry = process_chunk(j0, rows0, carry)
        prefetch(j0 + 2, idx0, rows0, sem_a)
        drain(rows1, sem_b)
        carry = process_chunk(j0 + 1, rows1, carry)
        return carry

    carry = lax.fori_loop(0, nch2 // 2, pair_body, (s0, jnp.int32(0), zero))
    drain(rows0, sem_a)
    lb = carry[1]

    def fill_empty(i, c):
        out_v[i, :] = bias
        return c

    lax.fori_loop(lb, bpw, fill_empty, jnp.int32(0))
    pltpu.sync_copy(out_v, out_hbm.at[pl.ds(base_bag, bpw)])


def _bag_pool(text, offs_ext, p_table, bias_pad, num_bags, n_tok):
    bpw = num_bags // NW
    offw = bpw + LANES
    mesh = plsc.VectorSubcoreMesh(
        core_axis_name="c", subcore_axis_name="s",
        num_cores=NC, num_subcores=NS)
    return pl.kernel(
        functools.partial(_bag_body, bpw, offw, n_tok),
        out_type=jax.ShapeDtypeStruct((num_bags, KP), jnp.float32),
        mesh=mesh,
        scratch_types=[
            pltpu.VMEM((offw,), jnp.int32),
            pltpu.VMEM((CH,), jnp.int32),
            pltpu.VMEM((CH,), jnp.int32),
            pltpu.VMEM((CH, KP), jnp.float32),
            pltpu.VMEM((CH, KP), jnp.float32),
            pltpu.VMEM((bpw, KP), jnp.float32),
            pltpu.VMEM((LANES,), jnp.float32),
            pltpu.SemaphoreType.DMA,
            pltpu.SemaphoreType.DMA,
        ],
        compiler_params=pltpu.CompilerParams(needs_layout_passes=False,
                                             use_tc_tiling_on_sc=False),
    )(text, offs_ext, p_table, bias_pad)


def kernel(text, offsets, emb_table, fc_w, fc_b):
    n_tokens = text.shape[0]
    num_bags = offsets.shape[0]
    k_classes = fc_w.shape[0]

    text = text.astype(jnp.int32)
    offsets = offsets.astype(jnp.int32)
    emb_table = emb_table.astype(jnp.float32)

    fc_wp = jnp.zeros((emb_table.shape[1], KP), jnp.float32)
    fc_wp = fc_wp.at[:, :k_classes].set(fc_w.astype(jnp.float32).T)
    bias_pad = jnp.zeros((KP,), jnp.float32)
    bias_pad = bias_pad.at[:k_classes].set(fc_b.astype(jnp.float32))

    p_table = _project_table(emb_table, fc_wp)

    bpw = num_bags // NW
    offs_ext = jnp.concatenate(
        [offsets, jnp.full((bpw + LANES,), n_tokens, jnp.int32)])

    out16 = _bag_pool(text, offs_ext, p_table, bias_pad, num_bags, n_tokens)
    return out16[:, :k_classes]


# speedup vs baseline: 1.0344x; 1.0344x over previous
import functools

import jax
import jax.numpy as jnp
from jax import lax
from jax.experimental import pallas as pl
from jax.experimental.pallas import tpu as pltpu
from jax.experimental.pallas import tpu_sc as plsc

NC = 2
NS = 16
NW = NC * NS
LANES = 16
KP = 16
CH = 2048
G = 128
NG = CH // G


def _project_table(emb_table, fc_wp):
    V, D = emb_table.shape
    grid = 1
    for cand in (8, 4, 2):
        if (V // 8) % cand == 0:
            grid = cand
            break
    rows = V // 8 // grid
    e8 = emb_table.reshape(grid, rows, 8 * D)
    wb = jnp.kron(jnp.eye(8, dtype=jnp.float32), fc_wp)

    def mm_body(x_ref, w_ref, o_ref):
        y = jnp.dot(x_ref[0], w_ref[...], preferred_element_type=jnp.float32)
        o_ref[...] = y.reshape(1, rows, 8 * KP)

    p3 = pl.pallas_call(
        mm_body,
        grid=(grid,),
        in_specs=[
            pl.BlockSpec((1, rows, 8 * D), lambda i: (i, 0, 0)),
            pl.BlockSpec((8 * D, 8 * KP), lambda i: (0, 0)),
        ],
        out_specs=pl.BlockSpec((1, rows, 8 * KP), lambda i: (i, 0, 0)),
        out_shape=jax.ShapeDtypeStruct((grid, rows, 8 * KP), jnp.float32),
    )(e8, wb)
    return p3.reshape(V, KP)


def _bag_body(bpw, offw, n_tok, text_hbm, offs_hbm, p_hbm, bias_hbm, out_hbm,
              off_v, idx0, idx1, rows0, rows1, out_v, bias_v, sem_a, sem_b):
    wid = lax.axis_index("s") * NC + lax.axis_index("c")
    base_bag = wid * bpw
    pltpu.sync_copy(offs_hbm.at[pl.ds(base_bag, offw)], off_v)
    pltpu.sync_copy(bias_hbm, bias_v)
    bias = bias_v[...]
    zero = jnp.zeros((LANES,), jnp.float32)

    s0 = off_v[pl.ds(0, LANES)][0]
    s1 = off_v[pl.ds(bpw, LANES)][0]
    a = jnp.bitwise_and(s0, jnp.int32(-8))
    nch = (s1 - a + (CH - 1)) // CH
    nch2 = ((nch + 1) // 2) * 2

    nvec = offw // LANES
    lane_iota = lax.iota(jnp.int32, LANES)

    def count_le(hi):
        m = jnp.int32(0)
        for k in range(nvec):
            valid = bpw + 1 - k * LANES
            if valid <= 0:
                break
            vk = off_v[pl.ds(k * LANES, LANES)]
            sel = (vk <= hi) & (lane_iota < valid)
            m = m + plsc.all_reduce_population_count(sel)[0]
        return m

    def chunk_start(j):
        c0 = jnp.minimum(a + j * CH, n_tok - CH)
        return pl.multiple_of(c0, 8)

    def prefetch(j, idx_b, rows_b, sem):
        c0 = chunk_start(j)
        pltpu.sync_copy(text_hbm.at[pl.ds(c0, CH)], idx_b)
        for g in range(NG):
            pltpu.async_copy(
                p_hbm.at[idx_b.at[pl.ds(g * G, G)]],
                rows_b.at[pl.ds(g * G, G)], sem)

    def drain(rows_b, sem):
        pltpu.make_async_copy(p_hbm.at[pl.ds(0, CH)], rows_b, sem).wait()

    def make_run_sum(rows_b):
        def run_sum(t0r, t1r, acc):
            n = jnp.maximum(t1r - t0r, 0)

            def tok8(i, st):
                a0, a1, a2, a3 = st
                b = t0r + i * 8
                a0 = a0 + rows_b[b, :]
                a1 = a1 + rows_b[b + 1, :]
                a2 = a2 + rows_b[b + 2, :]
                a3 = a3 + rows_b[b + 3, :]
                a0 = a0 + rows_b[b + 4, :]
                a1 = a1 + rows_b[b + 5, :]
                a2 = a2 + rows_b[b + 6, :]
                a3 = a3 + rows_b[b + 7, :]
                return a0, a1, a2, a3

            a0, a1, a2, a3 = lax.fori_loop(
                0, n // 8, tok8, (acc, zero, zero, zero))
            acc = (a0 + a1) + (a2 + a3)

            def tok1(i, a2_):
                return a2_ + rows_b[i, :]

            n8 = jnp.bitwise_and(n, jnp.int32(-8))
            return lax.fori_loop(t0r + n8, t1r, tok1, acc)

        return run_sum

    def process_chunk(j, rows_b, carry):
        t, lb, acc = carry
        run_sum = make_run_sum(rows_b)
        c0 = chunk_start(j)
        hi = jnp.minimum(c0 + CH, s1)
        nd = count_le(hi) - 1

        def bag_done(k, st):
            t, acc = st
            bv = off_v[pl.ds(k, LANES)]
            b_start, e_true = bv[0], bv[1]
            acc = run_sum(t - c0, e_true - c0, acc)
            cnt = (e_true - b_start).astype(jnp.float32)
            cnt_vec = jnp.full((LANES,), 1.0, jnp.float32) * cnt
            out_v[k, :] = acc / jnp.maximum(cnt_vec, 1.0) + bias
            return e_true, zero

        t, acc = lax.fori_loop(lb, nd, bag_done, (jnp.maximum(t, c0), acc))
        acc = run_sum(jnp.maximum(t, c0) - c0, hi - c0, acc)
        return hi, nd, acc

    prefetch(0, idx0, rows0, sem_a)

    def pair_body(jj, carry):
        j0 = 2 * jj
        prefetch(j0 + 1, idx1, rows1, sem_b)
        drain(rows0, sem_a)
        carry = process_chunk(j0, rows0, carry)
        prefetch(j0 + 2, idx0, rows0, sem_a)
        drain(rows1, sem_b)
        carry = process_chunk(j0 + 1, rows1, carry)
        return carry

    carry = lax.fori_loop(0, nch2 // 2, pair_body, (s0, jnp.int32(0), zero))
    drain(rows0, sem_a)
    lb = carry[1]

    def fill_empty(i, c):
        out_v[i, :] = bias
        return c

    lax.fori_loop(lb, bpw, fill_empty, jnp.int32(0))
    pltpu.sync_copy(out_v, out_hbm.at[pl.ds(base_bag, bpw)])


def _bag_pool(text, offs_ext, p_table, bias_pad, num_bags, n_tok):
    bpw = num_bags // NW
    offw = bpw + LANES
    mesh = plsc.VectorSubcoreMesh(
        core_axis_name="c", subcore_axis_name="s",
        num_cores=NC, num_subcores=NS)
    return pl.kernel(
        functools.partial(_bag_body, bpw, offw, n_tok),
        out_type=jax.ShapeDtypeStruct((num_bags, KP), jnp.float32),
        mesh=mesh,
        scratch_types=[
            pltpu.VMEM((offw,), jnp.int32),
            pltpu.VMEM((CH,), jnp.int32),
            pltpu.VMEM((CH,), jnp.int32),
            pltpu.VMEM((CH, KP), jnp.float32),
            pltpu.VMEM((CH, KP), jnp.float32),
            pltpu.VMEM((bpw, KP), jnp.float32),
            pltpu.VMEM((LANES,), jnp.float32),
            pltpu.SemaphoreType.DMA,
            pltpu.SemaphoreType.DMA,
        ],
        compiler_params=pltpu.CompilerParams(needs_layout_passes=False,
                                             use_tc_tiling_on_sc=False),
    )(text, offs_ext, p_table, bias_pad)


def kernel(text, offsets, emb_table, fc_w, fc_b):
    n_tokens = text.shape[0]
    num_bags = offsets.shape[0]
    k_classes = fc_w.shape[0]

    text = text.astype(jnp.int32)
    offsets = offsets.astype(jnp.int32)
    emb_table = emb_table.astype(jnp.float32)

    fc_wp = jnp.zeros((emb_table.shape[1], KP), jnp.float32)
    fc_wp = fc_wp.at[:, :k_classes].set(fc_w.astype(jnp.float32).T)
    bias_pad = jnp.zeros((KP,), jnp.float32)
    bias_pad = bias_pad.at[:k_classes].set(fc_b.astype(jnp.float32))

    p_table = _project_table(emb_table, fc_wp)

    bpw = num_bags // NW
    offs_ext = jnp.concatenate(
        [offsets, jnp.full((bpw + LANES,), n_tokens, jnp.int32)])

    out16 = _bag_pool(text, offs_ext, p_table, bias_pad, num_bags, n_tokens)
    return out16[:, :k_classes]


# speedup vs baseline: 1.0742x; 1.0385x over previous
import functools

import jax
import jax.numpy as jnp
from jax import lax
from jax.experimental import pallas as pl
from jax.experimental.pallas import tpu as pltpu
from jax.experimental.pallas import tpu_sc as plsc

NC = 2
NS = 16
NW = NC * NS
LANES = 16
KP = 16
CH = 2048
G = 128
NG = CH // G


def _project_table(emb_table, fc_wp):
    V, D = emb_table.shape
    grid = 1
    for cand in (8, 4, 2):
        if (V // 8) % cand == 0:
            grid = cand
            break
    rows = V // 8 // grid
    e8 = emb_table.astype(jnp.bfloat16).reshape(grid, rows, 8 * D)
    wb = jnp.kron(jnp.eye(8, dtype=jnp.float32), fc_wp).astype(jnp.bfloat16)

    def mm_body(x_ref, w_ref, o_ref):
        y = jnp.dot(x_ref[0], w_ref[...], preferred_element_type=jnp.float32)
        o_ref[...] = y.reshape(1, rows, 8 * KP)

    p3 = pl.pallas_call(
        mm_body,
        grid=(grid,),
        in_specs=[
            pl.BlockSpec((1, rows, 8 * D), lambda i: (i, 0, 0)),
            pl.BlockSpec((8 * D, 8 * KP), lambda i: (0, 0)),
        ],
        out_specs=pl.BlockSpec((1, rows, 8 * KP), lambda i: (i, 0, 0)),
        out_shape=jax.ShapeDtypeStruct((grid, rows, 8 * KP), jnp.float32),
    )(e8, wb)
    return p3.reshape(V, KP)


def _bag_body(bpw, offw, n_tok, text_hbm, offs_hbm, p_hbm, bias_hbm, out_hbm,
              off_v, idx0, idx1, rows0, rows1, out_v, bias_v, sem_a, sem_b):
    wid = lax.axis_index("s") * NC + lax.axis_index("c")
    base_bag = wid * bpw
    pltpu.sync_copy(offs_hbm.at[pl.ds(base_bag, offw)], off_v)
    pltpu.sync_copy(bias_hbm, bias_v)
    bias = bias_v[...]
    zero = jnp.zeros((LANES,), jnp.float32)

    s0 = off_v[pl.ds(0, LANES)][0]
    s1 = off_v[pl.ds(bpw, LANES)][0]
    a = jnp.bitwise_and(s0, jnp.int32(-8))
    nch = (s1 - a + (CH - 1)) // CH
    nch2 = ((nch + 1) // 2) * 2

    nvec = offw // LANES
    lane_iota = lax.iota(jnp.int32, LANES)

    def count_le(hi):
        m = jnp.int32(0)
        for k in range(nvec):
            valid = bpw + 1 - k * LANES
            if valid <= 0:
                break
            vk = off_v[pl.ds(k * LANES, LANES)]
            sel = (vk <= hi) & (lane_iota < valid)
            m = m + plsc.all_reduce_population_count(sel)[0]
        return m

    def chunk_start(j):
        c0 = jnp.minimum(a + j * CH, n_tok - CH)
        return pl.multiple_of(c0, 8)

    def prefetch(j, idx_b, rows_b, sem):
        c0 = chunk_start(j)
        pltpu.sync_copy(text_hbm.at[pl.ds(c0, CH)], idx_b)
        for g in range(NG):
            pltpu.async_copy(
                p_hbm.at[idx_b.at[pl.ds(g * G, G)]],
                rows_b.at[pl.ds(g * G, G)], sem)

    def drain(rows_b, sem):
        pltpu.make_async_copy(p_hbm.at[pl.ds(0, CH)], rows_b, sem).wait()

    def make_run_sum(rows_b):
        def run_sum(t0r, t1r, acc):
            n = jnp.maximum(t1r - t0r, 0)

            def tok8(i, st):
                a0, a1, a2, a3 = st
                b = t0r + i * 8
                a0 = a0 + rows_b[b, :]
                a1 = a1 + rows_b[b + 1, :]
                a2 = a2 + rows_b[b + 2, :]
                a3 = a3 + rows_b[b + 3, :]
                a0 = a0 + rows_b[b + 4, :]
                a1 = a1 + rows_b[b + 5, :]
                a2 = a2 + rows_b[b + 6, :]
                a3 = a3 + rows_b[b + 7, :]
                return a0, a1, a2, a3

            a0, a1, a2, a3 = lax.fori_loop(
                0, n // 8, tok8, (acc, zero, zero, zero))
            acc = (a0 + a1) + (a2 + a3)

            def tok1(i, a2_):
                return a2_ + rows_b[i, :]

            n8 = jnp.bitwise_and(n, jnp.int32(-8))
            return lax.fori_loop(t0r + n8, t1r, tok1, acc)

        return run_sum

    def process_chunk(j, rows_b, carry):
        t, lb, acc = carry
        run_sum = make_run_sum(rows_b)
        c0 = chunk_start(j)
        hi = jnp.minimum(c0 + CH, s1)
        nd = count_le(hi) - 1

        def bag_done(k, st):
            t, acc = st
            bv = off_v[pl.ds(k, LANES)]
            b_start, e_true = bv[0], bv[1]
            acc = run_sum(t - c0, e_true - c0, acc)
            cnt = (e_true - b_start).astype(jnp.float32)
            cnt_vec = jnp.full((LANES,), 1.0, jnp.float32) * cnt
            out_v[k, :] = acc / jnp.maximum(cnt_vec, 1.0) + bias
            return e_true, zero

        t, acc = lax.fori_loop(lb, nd, bag_done, (jnp.maximum(t, c0), acc))
        acc = run_sum(jnp.maximum(t, c0) - c0, hi - c0, acc)
        return hi, nd, acc

    prefetch(0, idx0, rows0, sem_a)

    def pair_body(jj, carry):
        j0 = 2 * jj
        prefetch(j0 + 1, idx1, rows1, sem_b)
        drain(rows0, sem_a)
        carry = process_chunk(j0, rows0, carry)
        prefetch(j0 + 2, idx0, rows0, sem_a)
        drain(rows1, sem_b)
        carry = process_chunk(j0 + 1, rows1, carry)
        return carry

    carry = lax.fori_loop(0, nch2 // 2, pair_body, (s0, jnp.int32(0), zero))
    drain(rows0, sem_a)
    lb = carry[1]

    def fill_empty(i, c):
        out_v[i, :] = bias
        return c

    lax.fori_loop(lb, bpw, fill_empty, jnp.int32(0))
    pltpu.sync_copy(out_v, out_hbm.at[pl.ds(base_bag, bpw)])


def _bag_pool(text, offs_ext, p_table, bias_pad, num_bags, n_tok):
    bpw = num_bags // NW
    offw = bpw + LANES
    mesh = plsc.VectorSubcoreMesh(
        core_axis_name="c", subcore_axis_name="s",
        num_cores=NC, num_subcores=NS)
    return pl.kernel(
        functools.partial(_bag_body, bpw, offw, n_tok),
        out_type=jax.ShapeDtypeStruct((num_bags, KP), jnp.float32),
        mesh=mesh,
        scratch_types=[
            pltpu.VMEM((offw,), jnp.int32),
            pltpu.VMEM((CH,), jnp.int32),
            pltpu.VMEM((CH,), jnp.int32),
            pltpu.VMEM((CH, KP), jnp.float32),
            pltpu.VMEM((CH, KP), jnp.float32),
            pltpu.VMEM((bpw, KP), jnp.float32),
            pltpu.VMEM((LANES,), jnp.float32),
            pltpu.SemaphoreType.DMA,
            pltpu.SemaphoreType.DMA,
        ],
        compiler_params=pltpu.CompilerParams(needs_layout_passes=False,
                                             use_tc_tiling_on_sc=False),
    )(text, offs_ext, p_table, bias_pad)


def kernel(text, offsets, emb_table, fc_w, fc_b):
    n_tokens = text.shape[0]
    num_bags = offsets.shape[0]
    k_classes = fc_w.shape[0]

    text = text.astype(jnp.int32)
    offsets = offsets.astype(jnp.int32)
    emb_table = emb_table.astype(jnp.float32)

    fc_wp = jnp.zeros((emb_table.shape[1], KP), jnp.float32)
    fc_wp = fc_wp.at[:, :k_classes].set(fc_w.astype(jnp.float32).T)
    bias_pad = jnp.zeros((KP,), jnp.float32)
    bias_pad = bias_pad.at[:k_classes].set(fc_b.astype(jnp.float32))

    p_table = _project_table(emb_table, fc_wp)

    bpw = num_bags // NW
    offs_ext = jnp.concatenate(
        [offsets, jnp.full((bpw + LANES,), n_tokens, jnp.int32)])

    out16 = _bag_pool(text, offs_ext, p_table, bias_pad, num_bags, n_tokens)
    return out16[:, :k_classes]
